# Initial kernel scaffold; baseline (speedup 1.0000x reference)
import functools

import jax
import jax.numpy as jnp
from jax import lax
from jax.experimental import pallas as pl
from jax.experimental.pallas import tpu as pltpu
from jax.experimental.pallas import tpu_sc as plsc

F32 = jnp.float32

N_NODES = 10000
D_IN = 128
T_STEPS = 50
HL = 32
G4 = 4 * HL
H = 16
OUT = 8
E_TOTAL = 320000

NB = 4
BLK = N_NODES // NB

NS = 16
CW = 128
EPT = E_TOTAL // NS
CN = -(-EPT // CW)
EPT_PAD = CN * CW
TAB_ROWS = N_NODES + 8
ACC_ROWS = 10016
STRIPE = ACC_ROWS // NS
DUMMY = N_NODES


def _enc_body(tsT, xcT, xsT, wih0, whh0, b0, wih1, whh1, b1,
              wxT, whT, bc, wsT, bs, outT, h1, c1, h2, c2):
    outT[1] = jnp.dot(wsT[:], xsT[:], preferred_element_type=F32) + bs[:]

    zero = jnp.zeros((HL, BLK), F32)
    h1[:] = zero
    c1[:] = zero
    h2[:] = zero
    c2[:] = zero

    wih0v = wih0[:]
    whh0v = whh0[:]
    b0v = b0[:]
    wih1v = wih1[:]
    whh1v = whh1[:]
    b1v = b1[:]

    def step(t, carry):
        x_t = tsT[t]
        g = (jnp.dot(wih0v, x_t, preferred_element_type=F32)
             + jnp.dot(whh0v, h1[:], preferred_element_type=F32) + b0v)
        i1 = jax.nn.sigmoid(g[0:HL])
        f1 = jax.nn.sigmoid(g[HL:2 * HL])
        gg1 = jnp.tanh(g[2 * HL:3 * HL])
        o1 = jax.nn.sigmoid(g[3 * HL:4 * HL])
        cn1 = f1 * c1[:] + i1 * gg1
        hn1 = o1 * jnp.tanh(cn1)
        c1[:] = cn1
        h1[:] = hn1

        g2 = (jnp.dot(wih1v, hn1, preferred_element_type=F32)
              + jnp.dot(whh1v, h2[:], preferred_element_type=F32) + b1v)
        i2 = jax.nn.sigmoid(g2[0:HL])
        f2 = jax.nn.sigmoid(g2[HL:2 * HL])
        gg2 = jnp.tanh(g2[2 * HL:3 * HL])
        o2 = jax.nn.sigmoid(g2[3 * HL:4 * HL])
        cn2 = f2 * c2[:] + i2 * gg2
        hn2 = o2 * jnp.tanh(cn2)
        c2[:] = cn2
        h2[:] = hn2
        return 0

    lax.fori_loop(0, T_STEPS, step, 0)

    outT[0] = (jnp.dot(wxT[:], xcT[:], preferred_element_type=F32)
               + jnp.dot(whT[:], h2[:], preferred_element_type=F32) + bc[:])


def _encoder(tsT, xcT, xsT, wih0, whh0, b0, wih1, whh1, b1,
             wxT, whT, bc, wsT, bs):
    full = lambda s: pl.BlockSpec(s, lambda i: tuple(0 for _ in s))
    return pl.pallas_call(
        _enc_body,
        grid=(NB,),
        in_specs=[
            pl.BlockSpec((T_STEPS, 6, BLK), lambda i: (0, 0, i)),
            pl.BlockSpec((D_IN, BLK), lambda i: (0, i)),
            pl.BlockSpec((D_IN, BLK), lambda i: (0, i)),
            full((G4, 6)), full((G4, HL)), full((G4, 1)),
            full((G4, HL)), full((G4, HL)), full((G4, 1)),
            full((H, D_IN)), full((H, HL)), full((H, 1)),
            full((H, D_IN)), full((H, 1)),
        ],
        out_specs=pl.BlockSpec((2, H, BLK), lambda i: (0, 0, i)),
        out_shape=jax.ShapeDtypeStruct((2, H, N_NODES), F32),
        scratch_shapes=[pltpu.VMEM((HL, BLK), F32) for _ in range(4)],
    )(tsT, xcT, xsT, wih0, whh0, b0, wih1, whh1, b1, wxT, whT, bc, wsT, bs)


_SC_MESH = plsc.VectorSubcoreMesh(core_axis_name="c", subcore_axis_name="s")


@functools.partial(
    pl.kernel,
    out_type=jax.ShapeDtypeStruct((2, ACC_ROWS, H), F32),
    mesh=_SC_MESH,
    scratch_types=[
        pltpu.VMEM((CN, CW), jnp.int32),
        pltpu.VMEM((CN, CW), jnp.int32),
        pltpu.VMEM((CW, H), F32),
        pltpu.VMEM((CW, H), F32),
        pltpu.VMEM((STRIPE, H), F32),
        pltpu.VMEM((STRIPE, H), F32),
        pltpu.VMEM_SHARED((ACC_ROWS, H), F32),
        pltpu.VMEM_SHARED((ACC_ROWS, H), F32),
        pltpu.SemaphoreType.DMA,
    ],
)
def _sc_seg(tab_c, tab_s, ec, es, ones_hbm, zeros_hbm, out_hbm,
            src_v, dst_v, rows_v, ones_v, acc_v, cnt_v, acc_sh, cnt_sh, sem):
    cid = lax.axis_index("c")
    sid = lax.axis_index("s")

    pltpu.sync_copy(zeros_hbm, acc_sh.at[pl.ds(sid * STRIPE, STRIPE)])
    pltpu.sync_copy(zeros_hbm, cnt_sh.at[pl.ds(sid * STRIPE, STRIPE)])
    pltpu.sync_copy(ones_hbm, ones_v)

    def accumulate(tab, edges):
        pltpu.sync_copy(edges.at[0, sid], src_v)
        pltpu.sync_copy(edges.at[1, sid], dst_v)

        def chunk(k, carry):
            pltpu.async_copy(tab.at[src_v.at[k]], rows_v, sem).wait()
            pltpu.sync_copy(rows_v, acc_sh.at[dst_v.at[k]], add=True)
            pltpu.sync_copy(ones_v, cnt_sh.at[dst_v.at[k]], add=True)
            return 0

        lax.fori_loop(0, CN, chunk, 0)

    plsc.subcore_barrier()

    @pl.when(cid == 0)
    def _():
        accumulate(tab_c, ec)

    @pl.when(cid == 1)
    def _():
        accumulate(tab_s, es)

    plsc.subcore_barrier()

    pltpu.sync_copy(acc_sh.at[pl.ds(sid * STRIPE, STRIPE)], acc_v)
    pltpu.sync_copy(cnt_sh.at[pl.ds(sid * STRIPE, STRIPE)], cnt_v)

    def divrow(i, carry):
        acc_v[i] = acc_v[i] / jnp.maximum(cnt_v[i], 1.0)
        return 0

    lax.fori_loop(0, STRIPE, divrow, 0)
    pltpu.sync_copy(acc_v, out_hbm.at[cid, pl.ds(sid * STRIPE, STRIPE)])


def _fin_body(mean_ref, xp_ref, a_ref, b_ref, c_ref, out_ref):
    val = (jnp.dot(mean_ref[0, 0:N_NODES, :], a_ref[0],
                   preferred_element_type=F32)
           + jnp.dot(xp_ref[0], b_ref[0], preferred_element_type=F32)
           + c_ref[0])
    out_ref[0] = jnp.maximum(val, 0.0)


def _finisher(means, xp, a_st, b_st, c_st):
    return pl.pallas_call(
        _fin_body,
        grid=(2,),
        in_specs=[
            pl.BlockSpec((1, ACC_ROWS, H), lambda i: (i, 0, 0)),
            pl.BlockSpec((1, N_NODES, H), lambda i: (i, 0, 0)),
            pl.BlockSpec((1, H, OUT), lambda i: (i, 0, 0)),
            pl.BlockSpec((1, H, OUT), lambda i: (i, 0, 0)),
            pl.BlockSpec((1, 1, OUT), lambda i: (i, 0, 0)),
        ],
        out_specs=pl.BlockSpec((1, N_NODES, OUT), lambda i: (i, 0, 0)),
        out_shape=jax.ShapeDtypeStruct((2, N_NODES, OUT), F32),
    )(means, xp, a_st, b_st, c_st)


def _pad_edges(ei):
    er = ei.reshape(2, NS, EPT)
    pad = jnp.full((2, NS, EPT_PAD - EPT), DUMMY, jnp.int32)
    return jnp.concatenate([er, pad], axis=2).reshape(2, NS, CN, CW)


def kernel(x_channel, x_sensor, channel_time_series, edge_index_channel,
           edge_index_sensor, lstm_Wih0, lstm_Whh0, lstm_bih0, lstm_bhh0,
           lstm_Wih1, lstm_Whh1, lstm_bih1, lstm_bhh1, proj_channel_W,
           proj_channel_b, proj_sensor_W, proj_sensor_b, sage_channel_Wl,
           sage_channel_bl, sage_channel_Wr, sage_sensor_Wl, sage_sensor_bl,
           sage_sensor_Wr, lin_W, lin_b):
    tsT = jnp.transpose(channel_time_series, (1, 2, 0))
    xcT = x_channel.T
    xsT = x_sensor.T

    b0 = (lstm_bih0 + lstm_bhh0).reshape(G4, 1)
    b1 = (lstm_bih1 + lstm_bhh1).reshape(G4, 1)
    wxT = proj_channel_W[:D_IN].T
    whT = proj_channel_W[D_IN:].T
    bc = proj_channel_b.reshape(H, 1)
    wsT = proj_sensor_W.T
    bs = proj_sensor_b.reshape(H, 1)

    xpT = _encoder(tsT, xcT, xsT, lstm_Wih0, lstm_Whh0, b0,
                   lstm_Wih1, lstm_Whh1, b1, wxT, whT, bc, wsT, bs)
    xp = jnp.transpose(xpT, (0, 2, 1))

    zpad = jnp.zeros((TAB_ROWS - N_NODES, H), F32)
    tab_c = jnp.concatenate([xp[0], zpad], axis=0)
    tab_s = jnp.concatenate([xp[1], zpad], axis=0)
    ec4 = _pad_edges(edge_index_channel)
    es4 = _pad_edges(edge_index_sensor)
    ones_h = jnp.ones((CW, H), F32)
    zeros_h = jnp.zeros((STRIPE, H), F32)

    means = _sc_seg(tab_c, tab_s, ec4, es4, ones_h, zeros_h)

    a_st = jnp.stack([sage_channel_Wl @ lin_W, sage_sensor_Wl @ lin_W])
    b_st = jnp.stack([sage_channel_Wr @ lin_W, sage_sensor_Wr @ lin_W])
    c_st = jnp.stack([sage_channel_bl @ lin_W + lin_b,
                      sage_sensor_bl @ lin_W + lin_b]).reshape(2, 1, OUT)

    out = _finisher(means, xp, a_st, b_st, c_st)
    return (out[0], out[1])


# speedup vs baseline: 8.2262x; 8.2262x over previous
import functools

import jax
import jax.numpy as jnp
from jax import lax
from jax.experimental import pallas as pl
from jax.experimental.pallas import tpu as pltpu
from jax.experimental.pallas import tpu_sc as plsc

F32 = jnp.float32

N_NODES = 10000
D_IN = 128
T_STEPS = 50
HL = 32
G4 = 4 * HL
H = 16
OUT = 8
E_TOTAL = 320000

NB = 1
BLK = N_NODES // NB

NS = 16
CW = 128
EPT = E_TOTAL // NS
CN = -(-EPT // CW)
EPT_PAD = CN * CW
TAB_ROWS = N_NODES + 8
ACC_ROWS = 10112
STRIPE = ACC_ROWS // NS
DUMMY = N_NODES


def _enc_body(tsT, xcT, xsT, wih0, whh0, b0, wih1, whh1, b1,
              wxT, whT, bc, wsT, bs, outT, h1, c1, h2, c2):
    outT[1] = jnp.dot(wsT[:], xsT[:], preferred_element_type=F32) + bs[:]

    zero = jnp.zeros((HL, BLK), F32)
    h1[:] = zero
    c1[:] = zero
    h2[:] = zero
    c2[:] = zero

    wih0v = wih0[:]
    whh0v = whh0[:]
    b0v = b0[:]
    wih1v = wih1[:]
    whh1v = whh1[:]
    b1v = b1[:]

    def step(t, carry):
        x_t = tsT[t]
        g = (jnp.dot(wih0v, x_t, preferred_element_type=F32)
             + jnp.dot(whh0v, h1[:], preferred_element_type=F32) + b0v)
        i1 = jax.nn.sigmoid(g[0:HL])
        f1 = jax.nn.sigmoid(g[HL:2 * HL])
        gg1 = jnp.tanh(g[2 * HL:3 * HL])
        o1 = jax.nn.sigmoid(g[3 * HL:4 * HL])
        cn1 = f1 * c1[:] + i1 * gg1
        hn1 = o1 * jnp.tanh(cn1)
        c1[:] = cn1
        h1[:] = hn1

        g2 = (jnp.dot(wih1v, hn1, preferred_element_type=F32)
              + jnp.dot(whh1v, h2[:], preferred_element_type=F32) + b1v)
        i2 = jax.nn.sigmoid(g2[0:HL])
        f2 = jax.nn.sigmoid(g2[HL:2 * HL])
        gg2 = jnp.tanh(g2[2 * HL:3 * HL])
        o2 = jax.nn.sigmoid(g2[3 * HL:4 * HL])
        cn2 = f2 * c2[:] + i2 * gg2
        hn2 = o2 * jnp.tanh(cn2)
        c2[:] = cn2
        h2[:] = hn2
        return 0

    lax.fori_loop(0, T_STEPS, step, 0)

    outT[0] = (jnp.dot(wxT[:], xcT[:], preferred_element_type=F32)
               + jnp.dot(whT[:], h2[:], preferred_element_type=F32) + bc[:])


def _encoder(tsT, xcT, xsT, wih0, whh0, b0, wih1, whh1, b1,
             wxT, whT, bc, wsT, bs):
    full = lambda s: pl.BlockSpec(s, lambda i: tuple(0 for _ in s))
    return pl.pallas_call(
        _enc_body,
        grid=(NB,),
        in_specs=[
            pl.BlockSpec((T_STEPS, 6, BLK), lambda i: (0, 0, i)),
            pl.BlockSpec((D_IN, BLK), lambda i: (0, i)),
            pl.BlockSpec((D_IN, BLK), lambda i: (0, i)),
            full((G4, 6)), full((G4, HL)), full((G4, 1)),
            full((G4, HL)), full((G4, HL)), full((G4, 1)),
            full((H, D_IN)), full((H, HL)), full((H, 1)),
            full((H, D_IN)), full((H, 1)),
        ],
        out_specs=pl.BlockSpec((2, H, BLK), lambda i: (0, 0, i)),
        out_shape=jax.ShapeDtypeStruct((2, H, N_NODES), F32),
        scratch_shapes=[pltpu.VMEM((HL, BLK), F32) for _ in range(4)],
    )(tsT, xcT, xsT, wih0, whh0, b0, wih1, whh1, b1, wxT, whT, bc, wsT, bs)


@functools.cache
def _sc_seg_call():
  mesh = plsc.VectorSubcoreMesh(core_axis_name="c", subcore_axis_name="s",
                                num_cores=2, num_subcores=NS)
  return pl.kernel(
    _sc_seg_body,
    out_type=jax.ShapeDtypeStruct((2, ACC_ROWS, H), F32),
    mesh=mesh,
    scratch_types=[
        pltpu.VMEM((CW,), jnp.int32),
        pltpu.VMEM((CW,), jnp.int32),
        pltpu.VMEM((CW, H), F32),
        pltpu.VMEM((CW, H), F32),
        pltpu.VMEM((STRIPE, H), F32),
        pltpu.VMEM((STRIPE, H), F32),
        pltpu.VMEM_SHARED((ACC_ROWS, H), F32),
        pltpu.VMEM_SHARED((ACC_ROWS, H), F32),
        pltpu.SemaphoreType.DMA,
    ],
    compiler_params=pltpu.CompilerParams(use_tc_tiling_on_sc=False),
  )


def _sc_seg_body(tab_c, tab_s, ec, es, ones_hbm, zeros_hbm, out_hbm,
                 src_c, dst_c, rows_v, ones_v, acc_v, cnt_v,
                 acc_sh, cnt_sh, sem):
    cid = lax.axis_index("c")
    sid = lax.axis_index("s")

    pltpu.sync_copy(zeros_hbm, acc_sh.at[pl.ds(sid * STRIPE, STRIPE)])
    pltpu.sync_copy(zeros_hbm, cnt_sh.at[pl.ds(sid * STRIPE, STRIPE)])
    pltpu.sync_copy(ones_hbm, ones_v)

    def accumulate(tab, edges):
        def chunk(k, carry):
            pltpu.sync_copy(edges.at[0, sid, k], src_c)
            pltpu.sync_copy(edges.at[1, sid, k], dst_c)
            pltpu.async_copy(tab.at[src_c], rows_v, sem).wait()
            pltpu.sync_copy(rows_v, acc_sh.at[dst_c], add=True)
            pltpu.sync_copy(ones_v, cnt_sh.at[dst_c], add=True)
            return 0

        lax.fori_loop(0, CN, chunk, 0)

    plsc.subcore_barrier()

    @pl.when(cid == 0)
    def _():
        accumulate(tab_c, ec)

    @pl.when(cid == 1)
    def _():
        accumulate(tab_s, es)

    plsc.subcore_barrier()

    pltpu.sync_copy(acc_sh.at[pl.ds(sid * STRIPE, STRIPE)], acc_v)
    pltpu.sync_copy(cnt_sh.at[pl.ds(sid * STRIPE, STRIPE)], cnt_v)

    def divrow(i, carry):
        acc_v[i] = acc_v[i] / jnp.maximum(cnt_v[i], 1.0)
        return 0

    lax.fori_loop(0, STRIPE, divrow, 0)
    pltpu.sync_copy(acc_v, out_hbm.at[cid, pl.ds(sid * STRIPE, STRIPE)])


def _fin_body(mean_ref, xp_ref, a_ref, b_ref, c_ref, out_ref):
    val = (jnp.dot(mean_ref[0, 0:N_NODES, :], a_ref[0],
                   preferred_element_type=F32)
           + jnp.dot(xp_ref[0], b_ref[0], preferred_element_type=F32)
           + c_ref[0])
    out_ref[0] = jnp.maximum(val, 0.0)


def _finisher(means, xp, a_st, b_st, c_st):
    return pl.pallas_call(
        _fin_body,
        grid=(2,),
        in_specs=[
            pl.BlockSpec((1, ACC_ROWS, H), lambda i: (i, 0, 0)),
            pl.BlockSpec((1, N_NODES, H), lambda i: (i, 0, 0)),
            pl.BlockSpec((1, H, OUT), lambda i: (i, 0, 0)),
            pl.BlockSpec((1, H, OUT), lambda i: (i, 0, 0)),
            pl.BlockSpec((1, 1, OUT), lambda i: (i, 0, 0)),
        ],
        out_specs=pl.BlockSpec((1, N_NODES, OUT), lambda i: (i, 0, 0)),
        out_shape=jax.ShapeDtypeStruct((2, N_NODES, OUT), F32),
    )(means, xp, a_st, b_st, c_st)


def _pad_edges(ei):
    er = ei.reshape(2, NS, EPT)
    pad = jnp.full((2, NS, EPT_PAD - EPT), DUMMY, jnp.int32)
    return jnp.concatenate([er, pad], axis=2).reshape(2, NS, CN, CW)


def kernel(x_channel, x_sensor, channel_time_series, edge_index_channel,
           edge_index_sensor, lstm_Wih0, lstm_Whh0, lstm_bih0, lstm_bhh0,
           lstm_Wih1, lstm_Whh1, lstm_bih1, lstm_bhh1, proj_channel_W,
           proj_channel_b, proj_sensor_W, proj_sensor_b, sage_channel_Wl,
           sage_channel_bl, sage_channel_Wr, sage_sensor_Wl, sage_sensor_bl,
           sage_sensor_Wr, lin_W, lin_b):
    tsT = jnp.transpose(channel_time_series, (1, 2, 0))
    xcT = x_channel.T
    xsT = x_sensor.T

    b0 = (lstm_bih0 + lstm_bhh0).reshape(G4, 1)
    b1 = (lstm_bih1 + lstm_bhh1).reshape(G4, 1)
    wxT = proj_channel_W[:D_IN].T
    whT = proj_channel_W[D_IN:].T
    bc = proj_channel_b.reshape(H, 1)
    wsT = proj_sensor_W.T
    bs = proj_sensor_b.reshape(H, 1)

    xpT = _encoder(tsT, xcT, xsT, lstm_Wih0, lstm_Whh0, b0,
                   lstm_Wih1, lstm_Whh1, b1, wxT, whT, bc, wsT, bs)
    xp = jnp.transpose(xpT, (0, 2, 1))

    zpad = jnp.zeros((TAB_ROWS - N_NODES, H), F32)
    tab_c = jnp.concatenate([xp[0], zpad], axis=0)
    tab_s = jnp.concatenate([xp[1], zpad], axis=0)
    ec4 = _pad_edges(edge_index_channel)
    es4 = _pad_edges(edge_index_sensor)
    ones_h = jnp.ones((CW, H), F32)
    zeros_h = jnp.zeros((STRIPE, H), F32)

    means = _sc_seg_call()(tab_c, tab_s, ec4, es4, ones_h, zeros_h)

    a_st = jnp.stack([sage_channel_Wl @ lin_W, sage_sensor_Wl @ lin_W])
    b_st = jnp.stack([sage_channel_Wr @ lin_W, sage_sensor_Wr @ lin_W])
    c_st = jnp.stack([sage_channel_bl @ lin_W + lin_b,
                      sage_sensor_bl @ lin_W + lin_b]).reshape(2, 1, OUT)

    out = _finisher(means, xp, a_st, b_st, c_st)
    return (out[0], out[1])


# speedup vs baseline: 13.9942x; 1.7012x over previous
import functools

import jax
import jax.numpy as jnp
from jax import lax
from jax.experimental import pallas as pl
from jax.experimental.pallas import tpu as pltpu
from jax.experimental.pallas import tpu_sc as plsc

F32 = jnp.float32

N_NODES = 10000
D_IN = 128
T_STEPS = 50
HL = 32
G4 = 4 * HL
H = 16
OUT = 8
E_TOTAL = 320000

NB = 1
BLK = N_NODES // NB

NS = 16
CW = 128
EROWS = E_TOTAL // CW
CN_FULL = EROWS // NS
X_ROWS = EROWS - CN_FULL * NS
GSZ = 4
SETS = 3
NBUF = SETS * GSZ
NG = CN_FULL // GSZ
ACC_ROWS = 10112
STRIPE = ACC_ROWS // NS


def _enc_body(tsT, xcT, xsT, wih0, whh0, b0, wih1, whh1, b1,
              wxT, whT, bc, wsT, bs, outT, h1, c1, h2, c2):
    outT[1] = jnp.dot(wsT[:], xsT[:], preferred_element_type=F32) + bs[:]

    zero = jnp.zeros((HL, BLK), F32)
    h1[:] = zero
    c1[:] = zero
    h2[:] = zero
    c2[:] = zero

    wih0v = wih0[:]
    whh0v = whh0[:]
    b0v = b0[:]
    wih1v = wih1[:]
    whh1v = whh1[:]
    b1v = b1[:]

    def step(t, carry):
        x_t = tsT[t]
        g = (jnp.dot(wih0v, x_t, preferred_element_type=F32)
             + jnp.dot(whh0v, h1[:], preferred_element_type=F32) + b0v)
        i1 = jax.nn.sigmoid(g[0:HL])
        f1 = jax.nn.sigmoid(g[HL:2 * HL])
        gg1 = jnp.tanh(g[2 * HL:3 * HL])
        o1 = jax.nn.sigmoid(g[3 * HL:4 * HL])
        cn1 = f1 * c1[:] + i1 * gg1
        hn1 = o1 * jnp.tanh(cn1)
        c1[:] = cn1
        h1[:] = hn1

        g2 = (jnp.dot(wih1v, hn1, preferred_element_type=F32)
              + jnp.dot(whh1v, h2[:], preferred_element_type=F32) + b1v)
        i2 = jax.nn.sigmoid(g2[0:HL])
        f2 = jax.nn.sigmoid(g2[HL:2 * HL])
        gg2 = jnp.tanh(g2[2 * HL:3 * HL])
        o2 = jax.nn.sigmoid(g2[3 * HL:4 * HL])
        cn2 = f2 * c2[:] + i2 * gg2
        hn2 = o2 * jnp.tanh(cn2)
        c2[:] = cn2
        h2[:] = hn2
        return 0

    lax.fori_loop(0, T_STEPS, step, 0)

    outT[0] = (jnp.dot(wxT[:], xcT[:], preferred_element_type=F32)
               + jnp.dot(whT[:], h2[:], preferred_element_type=F32) + bc[:])


def _encoder(tsT, xcT, xsT, wih0, whh0, b0, wih1, whh1, b1,
             wxT, whT, bc, wsT, bs):
    full = lambda s: pl.BlockSpec(s, lambda i: tuple(0 for _ in s))
    return pl.pallas_call(
        _enc_body,
        grid=(NB,),
        in_specs=[
            pl.BlockSpec((T_STEPS, 6, BLK), lambda i: (0, 0, i)),
            pl.BlockSpec((D_IN, BLK), lambda i: (0, i)),
            pl.BlockSpec((D_IN, BLK), lambda i: (0, i)),
            full((G4, 6)), full((G4, HL)), full((G4, 1)),
            full((G4, HL)), full((G4, HL)), full((G4, 1)),
            full((H, D_IN)), full((H, HL)), full((H, 1)),
            full((H, D_IN)), full((H, 1)),
        ],
        out_specs=pl.BlockSpec((2, H, BLK), lambda i: (0, 0, i)),
        out_shape=jax.ShapeDtypeStruct((2, H, N_NODES), F32),
        scratch_shapes=[pltpu.VMEM((HL, BLK), F32) for _ in range(4)],
    )(tsT, xcT, xsT, wih0, whh0, b0, wih1, whh1, b1, wxT, whT, bc, wsT, bs)


@functools.cache
def _sc_seg_call():
  mesh = plsc.VectorSubcoreMesh(core_axis_name="c", subcore_axis_name="s",
                                num_cores=2, num_subcores=NS)
  return pl.kernel(
    _sc_seg_body,
    out_type=jax.ShapeDtypeStruct((2, ACC_ROWS, H), F32),
    mesh=mesh,
    scratch_types=[
        pltpu.VMEM((CN_FULL + 1, CW), jnp.int32),
        pltpu.VMEM((CN_FULL + 1, CW), jnp.int32),
        pltpu.VMEM((NBUF, CW, H), F32),
        pltpu.VMEM((CW, H), F32),
        pltpu.VMEM((CW, H), F32),
        pltpu.VMEM((STRIPE, H), F32),
        pltpu.VMEM((STRIPE, H), F32),
        pltpu.VMEM_SHARED((ACC_ROWS, H), F32),
        pltpu.VMEM_SHARED((ACC_ROWS, H), F32),
        pltpu.SemaphoreType.DMA,
        pltpu.SemaphoreType.DMA,
        pltpu.SemaphoreType.DMA,
        pltpu.SemaphoreType.DMA,
        pltpu.SemaphoreType.DMA,
        pltpu.SemaphoreType.DMA,
    ],
    compiler_params=pltpu.CompilerParams(use_tc_tiling_on_sc=False),
  )


def _sc_seg_body(tab_c, tab_s, e3c, e3s, ones_hbm, zeros_hbm, out_hbm,
                 src_all, dst_all, rows_b, rows_x, ones_v, acc_v, cnt_v,
                 acc_sh, cnt_sh, g0, g1, g2, s0, s1, s2):
    cid = lax.axis_index("c")
    sid = lax.axis_index("s")
    gat = (g0, g1, g2)
    sct = (s0, s1, s2)

    pltpu.sync_copy(zeros_hbm, acc_sh.at[pl.ds(sid * STRIPE, STRIPE)])
    pltpu.sync_copy(zeros_hbm, cnt_sh.at[pl.ds(sid * STRIPE, STRIPE)])
    pltpu.sync_copy(ones_hbm, ones_v)
    plsc.subcore_barrier()

    def accumulate(tab, e3):
        pltpu.sync_copy(e3.at[0, pl.ds(sid * CN_FULL, CN_FULL)],
                        src_all.at[pl.ds(0, CN_FULL)])
        pltpu.sync_copy(e3.at[1, pl.ds(sid * CN_FULL, CN_FULL)],
                        dst_all.at[pl.ds(0, CN_FULL)])

        @pl.when(sid < X_ROWS)
        def _():
            pltpu.sync_copy(e3.at[0, CN_FULL * NS + sid], src_all.at[CN_FULL])
            pltpu.sync_copy(e3.at[1, CN_FULL * NS + sid], dst_all.at[CN_FULL])
            pltpu.async_copy(tab.at[src_all.at[CN_FULL]], rows_x, gat[0]).wait()
            pltpu.sync_copy(rows_x, acc_sh.at[dst_all.at[CN_FULL]], add=True)
            pltpu.sync_copy(ones_v, cnt_sh.at[dst_all.at[CN_FULL]], add=True)

        for b in range(GSZ):
            pltpu.async_copy(tab.at[src_all.at[b]], rows_b.at[b], gat[0])

        def body(j, carry):
            for p in range(SETS):
                g = SETS * j + p
                sn = (p + 1) % SETS

                @pl.when(g >= 2)
                def _():
                    for b in range(GSZ):
                        pltpu.make_async_copy(
                            ones_hbm, rows_b.at[sn * GSZ + b], sct[sn]).wait()
                        pltpu.make_async_copy(
                            ones_hbm, rows_b.at[sn * GSZ + b], sct[sn]).wait()

                @pl.when(g + 1 <= NG - 1)
                def _():
                    for b in range(GSZ):
                        pltpu.async_copy(tab.at[src_all.at[(g + 1) * GSZ + b]],
                                         rows_b.at[sn * GSZ + b], gat[sn])

                for b in range(GSZ):
                    pltpu.make_async_copy(
                        ones_hbm, rows_b.at[p * GSZ + b], gat[p]).wait()

                for b in range(GSZ):
                    pltpu.async_copy(rows_b.at[p * GSZ + b],
                                     acc_sh.at[dst_all.at[g * GSZ + b]],
                                     sct[p], add=True)
                    pltpu.async_copy(ones_v,
                                     cnt_sh.at[dst_all.at[g * GSZ + b]],
                                     sct[p], add=True)
            return 0

        lax.fori_loop(0, NG // SETS, body, 0)

        for gg in (NG - 2, NG - 1):
            sp = gg % SETS
            for b in range(GSZ):
                pltpu.make_async_copy(
                    ones_hbm, rows_b.at[sp * GSZ + b], sct[sp]).wait()
                pltpu.make_async_copy(
                    ones_hbm, rows_b.at[sp * GSZ + b], sct[sp]).wait()

    @pl.when(cid == 0)
    def _():
        accumulate(tab_c, e3c)

    @pl.when(cid == 1)
    def _():
        accumulate(tab_s, e3s)

    plsc.subcore_barrier()

    pltpu.sync_copy(acc_sh.at[pl.ds(sid * STRIPE, STRIPE)], acc_v)
    pltpu.sync_copy(cnt_sh.at[pl.ds(sid * STRIPE, STRIPE)], cnt_v)

    def divrow(i, carry):
        acc_v[i] = acc_v[i] / jnp.maximum(cnt_v[i], 1.0)
        return 0

    lax.fori_loop(0, STRIPE, divrow, 0)
    pltpu.sync_copy(acc_v, out_hbm.at[cid, pl.ds(sid * STRIPE, STRIPE)])


def _fin_body(mean_ref, xp_ref, a_ref, b_ref, c_ref, out_ref):
    val = (jnp.dot(mean_ref[0, 0:N_NODES, :], a_ref[0],
                   preferred_element_type=F32)
           + jnp.dot(xp_ref[0], b_ref[0], preferred_element_type=F32)
           + c_ref[0])
    out_ref[0] = jnp.maximum(val, 0.0)


def _finisher(means, xp, a_st, b_st, c_st):
    return pl.pallas_call(
        _fin_body,
        grid=(2,),
        in_specs=[
            pl.BlockSpec((1, ACC_ROWS, H), lambda i: (i, 0, 0)),
            pl.BlockSpec((1, N_NODES, H), lambda i: (i, 0, 0)),
            pl.BlockSpec((1, H, OUT), lambda i: (i, 0, 0)),
            pl.BlockSpec((1, H, OUT), lambda i: (i, 0, 0)),
            pl.BlockSpec((1, 1, OUT), lambda i: (i, 0, 0)),
        ],
        out_specs=pl.BlockSpec((1, N_NODES, OUT), lambda i: (i, 0, 0)),
        out_shape=jax.ShapeDtypeStruct((2, N_NODES, OUT), F32),
    )(means, xp, a_st, b_st, c_st)


def kernel(x_channel, x_sensor, channel_time_series, edge_index_channel,
           edge_index_sensor, lstm_Wih0, lstm_Whh0, lstm_bih0, lstm_bhh0,
           lstm_Wih1, lstm_Whh1, lstm_bih1, lstm_bhh1, proj_channel_W,
           proj_channel_b, proj_sensor_W, proj_sensor_b, sage_channel_Wl,
           sage_channel_bl, sage_channel_Wr, sage_sensor_Wl, sage_sensor_bl,
           sage_sensor_Wr, lin_W, lin_b):
    tsT = jnp.transpose(channel_time_series, (1, 2, 0))
    xcT = x_channel.T
    xsT = x_sensor.T

    b0 = (lstm_bih0 + lstm_bhh0).reshape(G4, 1)
    b1 = (lstm_bih1 + lstm_bhh1).reshape(G4, 1)
    wxT = proj_channel_W[:D_IN].T
    whT = proj_channel_W[D_IN:].T
    bc = proj_channel_b.reshape(H, 1)
    wsT = proj_sensor_W.T
    bs = proj_sensor_b.reshape(H, 1)

    xpT = _encoder(tsT, xcT, xsT, lstm_Wih0, lstm_Whh0, b0,
                   lstm_Wih1, lstm_Whh1, b1, wxT, whT, bc, wsT, bs)
    xp = jnp.transpose(xpT, (0, 2, 1))

    e3c = edge_index_channel.reshape(2, EROWS, CW)
    e3s = edge_index_sensor.reshape(2, EROWS, CW)
    ones_h = jnp.ones((CW, H), F32)
    zeros_h = jnp.zeros((STRIPE, H), F32)

    means = _sc_seg_call()(xp[0], xp[1], e3c, e3s, ones_h, zeros_h)

    a_st = jnp.stack([sage_channel_Wl @ lin_W, sage_sensor_Wl @ lin_W])
    b_st = jnp.stack([sage_channel_Wr @ lin_W, sage_sensor_Wr @ lin_W])
    c_st = jnp.stack([sage_channel_bl @ lin_W + lin_b,
                      sage_sensor_bl @ lin_W + lin_b]).reshape(2, 1, OUT)

    out = _finisher(means, xp, a_st, b_st, c_st)
    return (out[0], out[1])
